# baseline (speedup 1.0000x reference)
import functools

import jax
import jax.numpy as jnp
from jax import lax
from jax.experimental import pallas as pl
from jax.experimental.pallas import tpu as pltpu
from jax.experimental.pallas import tpu_sc as plsc

NC = 2
NS = 16
NW = NC * NS
ROWS = 65536
COLS = 100
MCOLS = 10
K = ROWS // MCOLS + 1
KSPLIT = 4096
KPW = KSPLIT // NW
NG = KPW // 16
BK = 512
KTOP = 6656
NB2 = (KTOP - KSPLIT) // BK

_OWNERS = [sorted({(MCOLS * l + m) // 16 for l in range(16)})
           for m in range(MCOLS)]


def _sc_partials(xt):
    mesh = plsc.VectorSubcoreMesh(core_axis_name="c", subcore_axis_name="s")

    @functools.partial(
        pl.kernel,
        out_type=jax.ShapeDtypeStruct((NW, 16), jnp.float32),
        mesh=mesh,
        scratch_types=[
            pltpu.VMEM((COLS, KPW), jnp.float32),
            pltpu.VMEM((16, KPW * MCOLS), jnp.float32),
            pltpu.VMEM((16,), jnp.float32),
        ],
    )
    def k(xt_hbm, out_hbm, a_v, b_v, res_v):
        wid = lax.axis_index("s") * NC + lax.axis_index("c")
        kw0 = wid * KPW
        pltpu.sync_copy(xt_hbm.at[:, pl.ds(kw0, KPW)], a_v)
        pltpu.sync_copy(
            xt_hbm.at[pl.ds(0, 16), pl.ds(kw0 * MCOLS, KPW * MCOLS)], b_v)

        iota = lax.iota(jnp.int32, 16)
        tbase = iota * MCOLS
        idx = [(tbase + m) & 15 for m in range(MCOLS)]
        tsel = [(tbase + m) >> 4 for m in range(MCOLS)]
        zero = jnp.zeros((16,), jnp.float32)

        def grp(g, acc_g):
            aoff = 16 * g
            boff = 160 * g

            def jrow(j, acc_j):
                w = [b_v[j, pl.ds(boff + 16 * t, 16)] for t in range(MCOLS)]
                acc2 = acc_j
                for m in range(MCOLS):
                    parts = [jnp.where(tsel[m] == t, w[t][idx[m]], zero)
                             for t in _OWNERS[m]]
                    while len(parts) > 1:
                        parts = [parts[i] + parts[i + 1]
                                 if i + 1 < len(parts) else parts[i]
                                 for i in range(0, len(parts), 2)]
                    va = a_v[MCOLS * m + j, pl.ds(aoff, 16)]
                    acc2 = acc2 + jnp.abs(va - parts[0])
                return acc2

            return lax.fori_loop(0, MCOLS, jrow, acc_g)

        acc = lax.fori_loop(0, NG, grp, jnp.zeros((16,), jnp.float32))
        res_v[...] = acc
        pltpu.sync_copy(res_v, out_hbm.at[wid])

    return k(xt)


def _tc_partials(xt):
    def body(a_ref, b_ref, o_ref):
        t = pl.program_id(0)
        a = a_ref[...]
        v10 = b_ref[pl.ds(0, MCOLS), :]
        b = (v10.reshape(MCOLS, BK, MCOLS)
             .transpose(2, 0, 1)
             .reshape(COLS, BK))
        kglob = (KSPLIT + BK * t
                 + lax.broadcasted_iota(jnp.int32, (COLS, BK), 1))
        cidx = lax.broadcasted_iota(jnp.int32, (COLS, BK), 0)
        valid = (kglob < K - 1) | ((kglob == K - 1) & (cidx < 60))
        d = jnp.where(valid, jnp.abs(a - b), 0.0)
        col = jnp.sum(d, axis=0).reshape(BK // 128, 128)
        o_ref[...] = jnp.concatenate(
            [col, jnp.zeros((8 - BK // 128, 128), jnp.float32)], axis=0)

    return pl.pallas_call(
        body,
        grid_spec=pl.GridSpec(
            grid=(NB2,),
            in_specs=[
                pl.BlockSpec((COLS, BK),
                             lambda t: (0, KSPLIT // BK + t)),
                pl.BlockSpec((16, BK * MCOLS),
                             lambda t: (0, KSPLIT // BK + t)),
            ],
            out_specs=pl.BlockSpec((8, 128), lambda t: (t, 0)),
        ),
        out_shape=jax.ShapeDtypeStruct((NB2 * 8, 128), jnp.float32),
    )(xt, xt)


def kernel(x):
    xt = x.T
    sc = _sc_partials(xt)
    tc = _tc_partials(xt)
    return jnp.sum(sc) + jnp.sum(tc)


# speedup vs baseline: 1.1916x; 1.1916x over previous
import functools

import jax
import jax.numpy as jnp
from jax import lax
from jax.experimental import pallas as pl
from jax.experimental.pallas import tpu as pltpu
from jax.experimental.pallas import tpu_sc as plsc

NC = 2
NS = 16
NW = NC * NS
ROWS = 65536
COLS = 100
MCOLS = 10
K = ROWS // MCOLS + 1
KSPLIT = 4096
KPW = KSPLIT // NW
NG = KPW // 16
BK = 256
KTOP = 6656
NB2 = (KTOP - KSPLIT) // BK

_OWNERS = [sorted({(MCOLS * l + m) // 16 for l in range(16)})
           for m in range(MCOLS)]


def _sc_partials(xt):
    mesh = plsc.VectorSubcoreMesh(core_axis_name="c", subcore_axis_name="s")

    @functools.partial(
        pl.kernel,
        out_type=jax.ShapeDtypeStruct((NW, 16), jnp.float32),
        mesh=mesh,
        scratch_types=[
            pltpu.VMEM((COLS, KPW), jnp.float32),
            pltpu.VMEM((16, KPW * MCOLS), jnp.float32),
            pltpu.VMEM((16,), jnp.float32),
        ],
    )
    def k(xt_hbm, out_hbm, a_v, b_v, res_v):
        wid = lax.axis_index("s") * NC + lax.axis_index("c")
        kw0 = wid * KPW
        pltpu.sync_copy(xt_hbm.at[:, pl.ds(kw0, KPW)], a_v)
        pltpu.sync_copy(
            xt_hbm.at[pl.ds(0, 16), pl.ds(kw0 * MCOLS, KPW * MCOLS)], b_v)

        iota = lax.iota(jnp.int32, 16)
        tbase = iota * MCOLS
        idx = [(tbase + m) & 15 for m in range(MCOLS)]
        tsel = [(tbase + m) >> 4 for m in range(MCOLS)]
        zero = jnp.zeros((16,), jnp.float32)

        def grp(g, acc_g):
            aoff = 16 * g
            boff = 160 * g
            acc2 = acc_g
            for j in range(MCOLS):
                w = [b_v[j, pl.ds(boff + 16 * t, 16)] for t in range(MCOLS)]
                for m in range(MCOLS):
                    parts = [jnp.where(tsel[m] == t, w[t][idx[m]], zero)
                             for t in _OWNERS[m]]
                    while len(parts) > 1:
                        parts = [parts[i] + parts[i + 1]
                                 if i + 1 < len(parts) else parts[i]
                                 for i in range(0, len(parts), 2)]
                    va = a_v[MCOLS * m + j, pl.ds(aoff, 16)]
                    acc2 = acc2 + jnp.abs(va - parts[0])
            return acc2

        acc = lax.fori_loop(0, NG, grp, jnp.zeros((16,), jnp.float32))
        res_v[...] = acc
        pltpu.sync_copy(res_v, out_hbm.at[wid])

    return k(xt)


def _tc_partials(xt):
    def body(a_ref, b_ref, o_ref):
        t = pl.program_id(0)
        a = a_ref[...]
        v10 = b_ref[pl.ds(0, MCOLS), :]
        b = (v10.reshape(MCOLS, BK, MCOLS)
             .transpose(2, 0, 1)
             .reshape(COLS, BK))
        kglob = (KSPLIT + BK * t
                 + lax.broadcasted_iota(jnp.int32, (COLS, BK), 1))
        cidx = lax.broadcasted_iota(jnp.int32, (COLS, BK), 0)
        valid = (kglob < K - 1) | ((kglob == K - 1) & (cidx < 60))
        d = jnp.where(valid, jnp.abs(a - b), 0.0)
        col = jnp.sum(d, axis=0).reshape(BK // 128, 128)
        o_ref[...] = jnp.concatenate(
            [col, jnp.zeros((8 - BK // 128, 128), jnp.float32)], axis=0)

    return pl.pallas_call(
        body,
        grid_spec=pl.GridSpec(
            grid=(NB2,),
            in_specs=[
                pl.BlockSpec((COLS, BK),
                             lambda t: (0, KSPLIT // BK + t)),
                pl.BlockSpec((16, BK * MCOLS),
                             lambda t: (0, KSPLIT // BK + t)),
            ],
            out_specs=pl.BlockSpec((8, 128), lambda t: (t, 0)),
        ),
        out_shape=jax.ShapeDtypeStruct((NB2 * 8, 128), jnp.float32),
    )(xt, xt)


def kernel(x):
    xt = x.T
    sc = _sc_partials(xt)
    tc = _tc_partials(xt)
    return jnp.sum(sc) + jnp.sum(tc)
